# Initial kernel scaffold; baseline (speedup 1.0000x reference)
import jax
import jax.numpy as jnp
from jax.experimental import pallas as pl


def kernel(x, edge_index, edge_weight, W0, as0, ad0, le0, ae0, b0, W1, as1, ad1, le1, ae1, b1, W2, as2, ad2, le2, ae2, b2):
    raise NotImplementedError("write your pallas kernel here")


# speedup vs baseline: 20.0972x; 20.0972x over previous
import functools

import jax
import jax.numpy as jnp
from jax import lax
from jax.experimental import pallas as pl
from jax.experimental.pallas import tpu as pltpu
from jax.experimental.pallas import tpu_sc as plsc

D = 128
NC = 2
NS = 16
NW = NC * NS
K = 128
L = 16


def _round_up(a, b):
    return (a + b - 1) // b * b


def _stats_body(ew_ref, s_ref, mx_ref, mn_ref):
    blk = ew_ref[...]
    s_ref[...] = jnp.sum(blk).reshape(1, 1)
    mx_ref[...] = jnp.max(blk).reshape(1, 1)
    mn_ref[...] = jnp.min(blk).reshape(1, 1)


def _edge_stats(ew):
    e = ew.shape[0]
    rows = e // D
    ew2 = ew.reshape(rows, D)
    return pl.pallas_call(
        _stats_body,
        out_shape=[jax.ShapeDtypeStruct((1, 1), jnp.float32)] * 3,
    )(ew2)


def _mm_body(x_ref, w_ref, asd_ref, le_ref, ae_ref,
             hx_ref, ad2_ref, mx_ref, ce_ref):
    i = pl.program_id(0)
    h = jnp.dot(x_ref[...], w_ref[...], preferred_element_type=jnp.float32)
    hx_ref[...] = h
    ad2 = jnp.dot(h, asd_ref[...], preferred_element_type=jnp.float32)
    ad2_ref[...] = ad2
    m = jnp.max(ad2[:, 0]).reshape(1, 1)

    @pl.when(i == 0)
    def _():
        mx_ref[...] = m
        ce_ref[...] = jnp.dot(le_ref[...], ae_ref[...],
                              preferred_element_type=jnp.float32)

    @pl.when(i != 0)
    def _():
        mx_ref[...] = jnp.maximum(mx_ref[...], m)


def _mm(x, W, a_s, a_d, le, ae, bn):
    n = x.shape[0]
    asd = jnp.stack([a_s, a_d], axis=1)
    ae2 = ae[:, None]
    return pl.pallas_call(
        _mm_body,
        grid=(n // bn,),
        in_specs=[
            pl.BlockSpec((bn, D), lambda i: (i, 0)),
            pl.BlockSpec((D, D), lambda i: (0, 0)),
            pl.BlockSpec((D, 2), lambda i: (0, 0)),
            pl.BlockSpec((1, D), lambda i: (0, 0)),
            pl.BlockSpec((D, 1), lambda i: (0, 0)),
        ],
        out_specs=[
            pl.BlockSpec((bn, D), lambda i: (i, 0)),
            pl.BlockSpec((bn, 2), lambda i: (i, 0)),
            pl.BlockSpec((1, 1), lambda i: (0, 0)),
            pl.BlockSpec((1, 1), lambda i: (0, 0)),
        ],
        out_shape=[
            jax.ShapeDtypeStruct((n, D), jnp.float32),
            jax.ShapeDtypeStruct((n, 2), jnp.float32),
            jax.ShapeDtypeStruct((1, 1), jnp.float32),
            jax.ShapeDtypeStruct((1, 1), jnp.float32),
        ],
    )(x, W, asd, le, ae2)


def _comb_resid_body(p_ref, d_ref, b_ref, xr_ref, y_ref):
    acc = p_ref[0] + p_ref[1]
    den = d_ref[0] + d_ref[1]
    y = acc / (den + 1e-16) + b_ref[...]
    y_ref[...] = jnp.maximum(y + xr_ref[...], 0.0)


def _comb_final_body(p_ref, d_ref, b_ref, y_ref):
    acc = p_ref[0] + p_ref[1]
    den = d_ref[0] + d_ref[1]
    y_ref[...] = acc / (den + 1e-16) + b_ref[...]


def _combine(outp, outd, b, xres, n, bn):
    b2 = b[None, :]
    np_ = outp.shape[1]
    outd2 = outd.reshape(NC, np_, 1)
    if xres is not None:
        return pl.pallas_call(
            _comb_resid_body,
            grid=(n // bn,),
            in_specs=[
                pl.BlockSpec((2, bn, D), lambda i: (0, i, 0)),
                pl.BlockSpec((2, bn, 1), lambda i: (0, i, 0)),
                pl.BlockSpec((1, D), lambda i: (0, 0)),
                pl.BlockSpec((bn, D), lambda i: (i, 0)),
            ],
            out_specs=pl.BlockSpec((bn, D), lambda i: (i, 0)),
            out_shape=jax.ShapeDtypeStruct((n, D), jnp.float32),
        )(outp, outd2, b2, xres)
    return pl.pallas_call(
        _comb_final_body,
        grid=(n // bn,),
        in_specs=[
            pl.BlockSpec((2, bn, D), lambda i: (0, i, 0)),
            pl.BlockSpec((2, bn, 1), lambda i: (0, i, 0)),
            pl.BlockSpec((1, D), lambda i: (0, 0)),
        ],
        out_specs=pl.BlockSpec((bn, D), lambda i: (i, 0)),
        out_shape=jax.ShapeDtypeStruct((n, D), jnp.float32),
    )(outp, outd2, b2)


def _build_edge_kernel(np_, g):
    rpt = np_ // NS

    mesh = plsc.VectorSubcoreMesh(core_axis_name="c", subcore_axis_name="s",
                                  num_cores=NC, num_subcores=NS)

    @functools.partial(
        pl.kernel,
        out_type=[jax.ShapeDtypeStruct((NC, np_, D), jnp.float32),
                  jax.ShapeDtypeStruct((NC, np_), jnp.float32)],
        mesh=mesh,
        scratch_types=[
            pltpu.VMEM((K,), jnp.int32),
            pltpu.VMEM((1, K), jnp.int32),
            pltpu.VMEM((K,), jnp.float32),
            pltpu.VMEM((K,), jnp.float32),
            pltpu.VMEM((K,), jnp.float32),
            pltpu.VMEM((K,), jnp.float32),
            pltpu.VMEM((K, D), jnp.float32),
            pltpu.VMEM((2 * L,), jnp.float32),
            pltpu.VMEM_SHARED((np_, D), jnp.float32),
            pltpu.VMEM_SHARED((np_,), jnp.float32),
            pltpu.SemaphoreType.DMA,
            pltpu.SemaphoreType.DMA,
            pltpu.SemaphoreType.DMA,
        ],
    )
    def edge_kernel(src2, dst2, ea2, asrcp, adstp, hx, par, outp, outd,
                    src_v, dst_v, ea_v, asv, adv, ex_v, hbuf, par_v,
                    acc, den, sem, sem2, sem3):
        c = lax.axis_index("c")
        s = lax.axis_index("s")
        wid = c * NS + s

        pltpu.sync_copy(par, par_v)
        ce_v = par_v[pl.ds(0, L)]
        m_v = par_v[pl.ds(L, L)]

        zv = jnp.zeros((L,), jnp.float32)

        def _zrow(r, _):
            for k2 in range(D // L):
                hbuf[r, pl.ds(k2 * L, L)] = zv
            return 0

        lax.fori_loop(0, K, _zrow, 0)
        for j in range(K // L):
            ex_v[pl.ds(j * L, L)] = zv

        def _zacc(j, _):
            r0 = s * rpt + j * K
            pltpu.sync_copy(hbuf, acc.at[pl.ds(r0, K)])
            pltpu.sync_copy(ex_v, den.at[pl.ds(r0, K)])
            return 0

        lax.fori_loop(0, rpt // K, _zacc, 0)
        plsc.subcore_barrier()

        def _chunk(gi, _):
            row = wid * g + gi
            pltpu.sync_copy(src2.at[row], src_v)
            pltpu.sync_copy(dst2.at[row], dst_v.at[0])
            pltpu.sync_copy(ea2.at[row], ea_v)
            gch = pltpu.async_copy(hx.at[src_v], hbuf, sem)
            gcs = pltpu.async_copy(asrcp.at[src_v], asv, sem2)
            gcd = pltpu.async_copy(adstp.at[dst_v.at[0]], adv, sem3)
            gcs.wait()
            gcd.wait()
            for j in range(K // L):
                sl = pl.ds(j * L, L)
                al = asv[sl] + adv[sl] + ea_v[sl] * ce_v
                al = jnp.maximum(al, 0.2 * al)
                cst = adv[sl] + m_v
                cst = jnp.maximum(cst, 0.2 * cst)
                ex_v[sl] = jnp.exp(al - cst)
            gch.wait()

            def _sgrp(j, _):
                ex16 = ex_v[pl.ds(j * L, L)]
                r0 = j * L
                for e in range(L):
                    exb = jnp.full((L,), ex16[e])
                    for k2 in range(D // L):
                        sl2 = pl.ds(k2 * L, L)
                        hbuf[r0 + e, sl2] = hbuf[r0 + e, sl2] * exb
                return 0

            lax.fori_loop(0, K // L, _sgrp, 0)
            pltpu.sync_copy(hbuf, acc.at[dst_v.at[0]], add=True)
            pltpu.sync_copy(ex_v, den.at[dst_v.at[0]], add=True)
            return 0

        lax.fori_loop(0, g, _chunk, 0)
        plsc.subcore_barrier()

        def _drain(j, _):
            r0 = s * rpt + j * K
            pltpu.sync_copy(acc.at[pl.ds(r0, K)], hbuf)
            pltpu.sync_copy(hbuf, outp.at[c, pl.ds(r0, K)])
            pltpu.sync_copy(den.at[pl.ds(r0, K)], ex_v)
            pltpu.sync_copy(ex_v, outd.at[c, pl.ds(r0, K)])
            return 0

        lax.fori_loop(0, rpt // K, _drain, 0)

    return edge_kernel


def kernel(x, edge_index, edge_weight,
           W0, as0, ad0, le0, ae0, b0,
           W1, as1, ad1, le1, ae1, b1,
           W2, as2, ad2, le2, ae2, b2):
    n = x.shape[0]
    e = edge_weight.shape[0]
    bn = 1000 if n % 1000 == 0 else 8
    np_ = _round_up(n + 1, NS * K)
    et = e + n
    g = -(-et // (NW * K))
    ep = NW * K * g
    rows = ep // K

    s_sum, s_mx, s_mn = _edge_stats(edge_weight)
    mean = s_sum[0, 0] / e

    loop = jnp.arange(n, dtype=jnp.int32)
    pad = ep - et
    srcf = jnp.concatenate([edge_index[0], loop,
                            jnp.zeros((pad,), jnp.int32)])
    dstf = jnp.concatenate([edge_index[1], loop,
                            jnp.full((pad,), n, jnp.int32)])
    eaf = jnp.concatenate([edge_weight, jnp.full((n,), mean),
                           jnp.zeros((pad,), jnp.float32)])
    src2 = srcf.reshape(rows, K)
    dst2 = dstf.reshape(rows, K)
    ea2 = eaf.reshape(rows, K)

    edge_fn = _build_edge_kernel(np_, g)

    def layer(x_in, W, a_s, a_d, le, ae, b, resid):
        hx, ad2_, mxs, ce = _mm(x_in, W, a_s, a_d, le, ae, bn)
        ces = ce[0, 0]
        m = mxs[0, 0] + jnp.maximum(ces * s_mx[0, 0], ces * s_mn[0, 0])
        par = jnp.concatenate([jnp.full((L,), ces), jnp.full((L,), m)])
        asrcp = jnp.pad(ad2_[:, 0], (0, np_ - n))
        adstp = jnp.pad(ad2_[:, 1], (0, np_ - n))
        outp, outd = edge_fn(src2, dst2, ea2, asrcp, adstp, hx, par)
        return _combine(outp, outd, b, x_in if resid else None, n, bn)

    y = layer(x, W0, as0, ad0, le0, ae0, b0, True)
    y = layer(y, W1, as1, ad1, le1, ae1, b1, True)
    return layer(y, W2, as2, ad2, le2, ae2, b2, False)


# speedup vs baseline: 20.3113x; 1.0107x over previous
import functools

import jax
import jax.numpy as jnp
from jax import lax
from jax.experimental import pallas as pl
from jax.experimental.pallas import tpu as pltpu
from jax.experimental.pallas import tpu_sc as plsc

D = 128
NC = 2
NS = 16
NW = NC * NS
K = 128
L = 16


def _round_up(a, b):
    return (a + b - 1) // b * b


def _stats_body(ew_ref, s_ref, mx_ref, mn_ref):
    blk = ew_ref[...]
    s_ref[...] = jnp.sum(blk).reshape(1, 1)
    mx_ref[...] = jnp.max(blk).reshape(1, 1)
    mn_ref[...] = jnp.min(blk).reshape(1, 1)


def _edge_stats(ew):
    e = ew.shape[0]
    rows = e // D
    ew2 = ew.reshape(rows, D)
    return pl.pallas_call(
        _stats_body,
        out_shape=[jax.ShapeDtypeStruct((1, 1), jnp.float32)] * 3,
    )(ew2)


def _mm_body(x_ref, w_ref, asd_ref, le_ref, ae_ref,
             hx_ref, ad2_ref, mx_ref, ce_ref):
    i = pl.program_id(0)
    h = jnp.dot(x_ref[...], w_ref[...], preferred_element_type=jnp.float32)
    hx_ref[...] = h
    ad2 = jnp.dot(h, asd_ref[...], preferred_element_type=jnp.float32)
    ad2_ref[...] = ad2
    m = jnp.max(ad2[:, 0]).reshape(1, 1)

    @pl.when(i == 0)
    def _():
        mx_ref[...] = m
        ce_ref[...] = jnp.dot(le_ref[...], ae_ref[...],
                              preferred_element_type=jnp.float32)

    @pl.when(i != 0)
    def _():
        mx_ref[...] = jnp.maximum(mx_ref[...], m)


def _mm(x, W, a_s, a_d, le, ae, bn):
    n = x.shape[0]
    asd = jnp.stack([a_s, a_d], axis=1)
    ae2 = ae[:, None]
    return pl.pallas_call(
        _mm_body,
        grid=(n // bn,),
        in_specs=[
            pl.BlockSpec((bn, D), lambda i: (i, 0)),
            pl.BlockSpec((D, D), lambda i: (0, 0)),
            pl.BlockSpec((D, 2), lambda i: (0, 0)),
            pl.BlockSpec((1, D), lambda i: (0, 0)),
            pl.BlockSpec((D, 1), lambda i: (0, 0)),
        ],
        out_specs=[
            pl.BlockSpec((bn, D), lambda i: (i, 0)),
            pl.BlockSpec((bn, 2), lambda i: (i, 0)),
            pl.BlockSpec((1, 1), lambda i: (0, 0)),
            pl.BlockSpec((1, 1), lambda i: (0, 0)),
        ],
        out_shape=[
            jax.ShapeDtypeStruct((n, D), jnp.float32),
            jax.ShapeDtypeStruct((n, 2), jnp.float32),
            jax.ShapeDtypeStruct((1, 1), jnp.float32),
            jax.ShapeDtypeStruct((1, 1), jnp.float32),
        ],
    )(x, W, asd, le, ae2)


def _comb_resid_body(p_ref, d_ref, b_ref, xr_ref, y_ref):
    acc = p_ref[0] + p_ref[1]
    den = d_ref[0] + d_ref[1]
    y = acc / (den + 1e-16) + b_ref[...]
    y_ref[...] = jnp.maximum(y + xr_ref[...], 0.0)


def _comb_final_body(p_ref, d_ref, b_ref, y_ref):
    acc = p_ref[0] + p_ref[1]
    den = d_ref[0] + d_ref[1]
    y_ref[...] = acc / (den + 1e-16) + b_ref[...]


def _combine(outp, outd, b, xres, n, bn):
    b2 = b[None, :]
    np_ = outp.shape[1]
    outd2 = outd.reshape(NC, np_, 1)
    if xres is not None:
        return pl.pallas_call(
            _comb_resid_body,
            grid=(n // bn,),
            in_specs=[
                pl.BlockSpec((2, bn, D), lambda i: (0, i, 0)),
                pl.BlockSpec((2, bn, 1), lambda i: (0, i, 0)),
                pl.BlockSpec((1, D), lambda i: (0, 0)),
                pl.BlockSpec((bn, D), lambda i: (i, 0)),
            ],
            out_specs=pl.BlockSpec((bn, D), lambda i: (i, 0)),
            out_shape=jax.ShapeDtypeStruct((n, D), jnp.float32),
        )(outp, outd2, b2, xres)
    return pl.pallas_call(
        _comb_final_body,
        grid=(n // bn,),
        in_specs=[
            pl.BlockSpec((2, bn, D), lambda i: (0, i, 0)),
            pl.BlockSpec((2, bn, 1), lambda i: (0, i, 0)),
            pl.BlockSpec((1, D), lambda i: (0, 0)),
        ],
        out_specs=pl.BlockSpec((bn, D), lambda i: (i, 0)),
        out_shape=jax.ShapeDtypeStruct((n, D), jnp.float32),
    )(outp, outd2, b2)


def _build_edge_kernel(np_, g):
    rpt = np_ // NS

    mesh = plsc.VectorSubcoreMesh(core_axis_name="c", subcore_axis_name="s",
                                  num_cores=NC, num_subcores=NS)
    NB = 2

    @functools.partial(
        pl.kernel,
        out_type=[jax.ShapeDtypeStruct((NC, np_, D), jnp.float32),
                  jax.ShapeDtypeStruct((NC, np_), jnp.float32)],
        mesh=mesh,
        scratch_types=[
            pltpu.VMEM((NB, 2, K), jnp.int32),
            pltpu.VMEM((NB, K), jnp.float32),
            pltpu.VMEM((NB, K), jnp.float32),
            pltpu.VMEM((NB, K), jnp.float32),
            pltpu.VMEM((NB, K), jnp.float32),
            pltpu.VMEM((NB, K, D), jnp.float32),
            pltpu.VMEM((2 * L,), jnp.float32),
            pltpu.VMEM_SHARED((np_, D), jnp.float32),
            pltpu.VMEM_SHARED((np_,), jnp.float32),
        ] + [pltpu.SemaphoreType.DMA] * (4 * NB),
    )
    def edge_kernel(sd3, ea2, asrcp, adstp, hx, par, outp, outd,
                    sd_v, ea_v, asv, adv, ex_v, hbuf, par_v,
                    acc, den, *sems):
        c = lax.axis_index("c")
        s = lax.axis_index("s")
        wid = c * NS + s

        pltpu.sync_copy(par, par_v)
        ce_v = par_v[pl.ds(0, L)]
        m_v = par_v[pl.ds(L, L)]

        zv = jnp.zeros((L,), jnp.float32)

        def _zrow(r, _):
            for k2 in range(D // L):
                hbuf[0, r, pl.ds(k2 * L, L)] = zv
            return 0

        lax.fori_loop(0, K, _zrow, 0)
        for j in range(K // L):
            ex_v[0, pl.ds(j * L, L)] = zv

        def _zacc(j, _):
            r0 = s * rpt + j * K
            pltpu.sync_copy(hbuf.at[0], acc.at[pl.ds(r0, K)])
            pltpu.sync_copy(ex_v.at[0], den.at[pl.ds(r0, K)])
            return 0

        lax.fori_loop(0, rpt // K, _zacc, 0)
        plsc.subcore_barrier()

        def _issue(b, gi):
            row = wid * g + gi
            pltpu.sync_copy(sd3.at[row], sd_v.at[b])
            pltpu.sync_copy(ea2.at[row], ea_v.at[b])
            return (
                pltpu.async_copy(hx.at[sd_v.at[b, 0]], hbuf.at[b],
                                 sems[4 * b]),
                pltpu.async_copy(asrcp.at[sd_v.at[b, 0]], asv.at[b],
                                 sems[4 * b + 1]),
                pltpu.async_copy(adstp.at[sd_v.at[b, 1]], adv.at[b],
                                 sems[4 * b + 2]),
            )

        def _process(b, cps):
            gch, gcs, gcd = cps
            gcs.wait()
            gcd.wait()
            for j in range(K // L):
                sl = pl.ds(j * L, L)
                al = asv[b, sl] + adv[b, sl] + ea_v[b, sl] * ce_v
                al = jnp.maximum(al, 0.2 * al)
                cst = adv[b, sl] + m_v
                cst = jnp.maximum(cst, 0.2 * cst)
                ex_v[b, sl] = jnp.exp(al - cst)
            gch.wait()

            def _sgrp(j, _):
                ex16 = ex_v[b, pl.ds(j * L, L)]
                r0 = j * L
                for e in range(L):
                    exb = jnp.full((L,), ex16[e])
                    for k2 in range(D // L):
                        sl2 = pl.ds(k2 * L, L)
                        hbuf[b, r0 + e, sl2] = hbuf[b, r0 + e, sl2] * exb
                return 0

            lax.fori_loop(0, K // L, _sgrp, 0)
            pltpu.sync_copy(hbuf.at[b], acc.at[sd_v.at[b, 1]], add=True)
            pltpu.sync_copy(ex_v.at[b], den.at[sd_v.at[b, 1]], add=True)

        _issue(0, 0)

        def _pair(it, _):
            g0 = NB * it
            cp1 = _issue(1, g0 + 1)
            cp0 = (pltpu.make_async_copy(hx.at[sd_v.at[0, 0]], hbuf.at[0],
                                         sems[0]),
                   pltpu.make_async_copy(asrcp.at[sd_v.at[0, 0]], asv.at[0],
                                         sems[1]),
                   pltpu.make_async_copy(adstp.at[sd_v.at[0, 1]], adv.at[0],
                                         sems[2]))
            _process(0, cp0)

            @pl.when(it + 1 < g // NB)
            def _():
                _issue(0, g0 + 2)

            _process(1, cp1)
            return 0

        lax.fori_loop(0, g // NB, _pair, 0)
        plsc.subcore_barrier()

        def _drain(j, _):
            r0 = s * rpt + j * K
            pltpu.sync_copy(acc.at[pl.ds(r0, K)], hbuf.at[0])
            pltpu.sync_copy(hbuf.at[0], outp.at[c, pl.ds(r0, K)])
            pltpu.sync_copy(den.at[pl.ds(r0, K)], ex_v.at[0])
            pltpu.sync_copy(ex_v.at[0], outd.at[c, pl.ds(r0, K)])
            return 0

        lax.fori_loop(0, rpt // K, _drain, 0)

    return edge_kernel


def kernel(x, edge_index, edge_weight,
           W0, as0, ad0, le0, ae0, b0,
           W1, as1, ad1, le1, ae1, b1,
           W2, as2, ad2, le2, ae2, b2):
    n = x.shape[0]
    e = edge_weight.shape[0]
    bn = 1000 if n % 1000 == 0 else 8
    np_ = _round_up(n + 1, NS * K)
    et = e + n
    g = _round_up(-(-et // (NW * K)), 2)
    ep = NW * K * g
    rows = ep // K

    s_sum, s_mx, s_mn = _edge_stats(edge_weight)
    mean = s_sum[0, 0] / e

    loop = jnp.arange(n, dtype=jnp.int32)
    pad = ep - et
    srcf = jnp.concatenate([edge_index[0], loop,
                            jnp.zeros((pad,), jnp.int32)])
    dstf = jnp.concatenate([edge_index[1], loop,
                            jnp.full((pad,), n, jnp.int32)])
    eaf = jnp.concatenate([edge_weight, jnp.full((n,), mean),
                           jnp.zeros((pad,), jnp.float32)])
    sd3 = jnp.stack([srcf.reshape(rows, K), dstf.reshape(rows, K)], axis=1)
    ea2 = eaf.reshape(rows, K)

    edge_fn = _build_edge_kernel(np_, g)

    def layer(x_in, W, a_s, a_d, le, ae, b, resid):
        hx, ad2_, mxs, ce = _mm(x_in, W, a_s, a_d, le, ae, bn)
        ces = ce[0, 0]
        m = mxs[0, 0] + jnp.maximum(ces * s_mx[0, 0], ces * s_mn[0, 0])
        par = jnp.concatenate([jnp.full((L,), ces), jnp.full((L,), m)])
        asrcp = jnp.pad(ad2_[:, 0], (0, np_ - n))
        adstp = jnp.pad(ad2_[:, 1], (0, np_ - n))
        outp, outd = edge_fn(sd3, ea2, asrcp, adstp, hx, par)
        return _combine(outp, outd, b, x_in if resid else None, n, bn)

    y = layer(x, W0, as0, ad0, le0, ae0, b0, True)
    y = layer(y, W1, as1, ad1, le1, ae1, b1, True)
    return layer(y, W2, as2, ad2, le2, ae2, b2, False)
